# Initial kernel scaffold; baseline (speedup 1.0000x reference)
import jax
import jax.numpy as jnp
from jax.experimental import pallas as pl


def kernel(x, lut):
    raise NotImplementedError("write your pallas kernel here")


# speedup vs baseline: 1.3044x; 1.3044x over previous
import functools
import math

import jax
import jax.numpy as jnp
from jax import lax
from jax.experimental import pallas as pl
from jax.experimental.pallas import tpu as pltpu
from jax.experimental.pallas import tpu_sc as plsc

D_MODEL = 1024
VOCAB = 100000
SCALE = math.sqrt(D_MODEL)

NC = 2
NS = 16
LANES = 16
NW = NC * NS

B = 4 * 4096
B_PER_W = B // NW
CHUNK = 32
NCHUNK = B_PER_W // CHUNK


def _scale_rows(buf):
    def row_body(r, carry):
        for j in range(D_MODEL // LANES):
            s = pl.ds(j * LANES, LANES)
            buf[r, s] = buf[r, s] * SCALE
        return carry
    lax.fori_loop(0, CHUNK, row_body, 0)


@functools.partial(
    pl.kernel,
    out_type=jax.ShapeDtypeStruct((B, D_MODEL), jnp.float32),
    mesh=plsc.VectorSubcoreMesh(core_axis_name="c", subcore_axis_name="s"),
    scratch_types=[
        pltpu.VMEM((NCHUNK, CHUNK), jnp.int32),
        pltpu.VMEM((CHUNK, D_MODEL), jnp.float32),
        pltpu.VMEM((CHUNK, D_MODEL), jnp.float32),
        pltpu.SemaphoreType.DMA,
        pltpu.SemaphoreType.DMA,
    ],
)
def _emb_kernel(x_hbm, lut_hbm, out_hbm, idx_v, buf0, buf1, sem0, sem1):
    wid = lax.axis_index("s") * NC + lax.axis_index("c")
    base = wid * B_PER_W

    pltpu.sync_copy(x_hbm.at[wid], idx_v)

    bufs = (buf0, buf1)
    sems = (sem0, sem1)
    copies = [None, None]
    copies[0] = pltpu.async_copy(lut_hbm.at[idx_v.at[0]], buf0, sem0)
    for i in range(NCHUNK):
        cur = i % 2
        nxt = (i + 1) % 2
        if i + 1 < NCHUNK:
            copies[nxt] = pltpu.async_copy(
                lut_hbm.at[idx_v.at[i + 1]], bufs[nxt], sems[nxt])
        copies[cur].wait()
        _scale_rows(bufs[cur])
        pltpu.sync_copy(bufs[cur], out_hbm.at[pl.ds(base + i * CHUNK, CHUNK)])


def kernel(x, lut):
    x_flat = jnp.reshape(x.astype(jnp.int32), (NW, NCHUNK, CHUNK))
    out = _emb_kernel(x_flat, lut)
    return jnp.reshape(out, (x.shape[0], x.shape[1], D_MODEL))


# speedup vs baseline: 1.5202x; 1.1654x over previous
import functools
import math

import jax
import jax.numpy as jnp
from jax import lax
from jax.experimental import pallas as pl
from jax.experimental.pallas import tpu as pltpu
from jax.experimental.pallas import tpu_sc as plsc

D_MODEL = 1024
VOCAB = 100000
SCALE = math.sqrt(D_MODEL)

NC = 2
NS = 16
LANES = 16
NW = NC * NS

B = 4 * 4096
B_PER_W = B // NW
CHUNK = 8
NCHUNK = B_PER_W // CHUNK
NBUF = 4
NROUND = NCHUNK // NBUF


@functools.partial(
    pl.kernel,
    out_type=jax.ShapeDtypeStruct((B, D_MODEL), jnp.float32),
    mesh=plsc.VectorSubcoreMesh(core_axis_name="c", subcore_axis_name="s"),
    scratch_types=[
        pltpu.VMEM((NCHUNK, CHUNK), jnp.int32),
        [pltpu.VMEM((CHUNK, D_MODEL), jnp.float32) for _ in range(NBUF)],
        [pltpu.VMEM((CHUNK, D_MODEL), jnp.float32) for _ in range(NBUF)],
        [pltpu.SemaphoreType.DMA for _ in range(NBUF)],
        [pltpu.SemaphoreType.DMA for _ in range(NBUF)],
    ],
)
def _emb_kernel(x_hbm, lut_hbm, out_hbm, idx_v, inb, outb, gsem, ssem):
    wid = lax.axis_index("s") * NC + lax.axis_index("c")
    base = wid * B_PER_W

    pltpu.sync_copy(x_hbm.at[wid], idx_v)

    def gather(i, b):
        return pltpu.async_copy(lut_hbm.at[idx_v.at[i]], inb[b], gsem[b])

    def wait_gather(b):
        pltpu.make_async_copy(lut_hbm.at[idx_v.at[0]], inb[b], gsem[b]).wait()

    def scatter(i, b):
        return pltpu.async_copy(
            outb[b], out_hbm.at[pl.ds(base + i * CHUNK, CHUNK)], ssem[b])

    def wait_scatter(b):
        pltpu.make_async_copy(
            outb[b], out_hbm.at[pl.ds(base, CHUNK)], ssem[b]).wait()

    def scale(b):
        def row_body(r, carry):
            for j in range(D_MODEL // LANES):
                s = pl.ds(j * LANES, LANES)
                outb[b][r, s] = inb[b][r, s] * SCALE
            return carry
        lax.fori_loop(0, CHUNK, row_body, 0)

    for b in range(NBUF):
        gather(b, b)

    for b in range(NBUF):
        wait_gather(b)
        scale(b)
        scatter(b, b)
        gather(NBUF + b, b)

    def round_body(g, carry):
        i0 = g * NBUF
        for b in range(NBUF):
            wait_gather(b)
            wait_scatter(b)
            scale(b)
            scatter(i0 + b, b)
            gather(i0 + NBUF + b, b)
        return carry
    lax.fori_loop(1, NROUND - 1, round_body, 0)

    i0 = (NROUND - 1) * NBUF
    for b in range(NBUF):
        wait_gather(b)
        wait_scatter(b)
        scale(b)
        scatter(i0 + b, b)

    for b in range(NBUF):
        wait_scatter(b)


def kernel(x, lut):
    x_flat = jnp.reshape(x.astype(jnp.int32), (NW, NCHUNK, CHUNK))
    out = _emb_kernel(x_flat, lut)
    return jnp.reshape(out, (x.shape[0], x.shape[1], D_MODEL))
